# baseline (speedup 1.0000x reference)
import functools

import jax
import jax.numpy as jnp
from jax import lax
from jax.experimental import pallas as pl
from jax.experimental.pallas import tpu as pltpu
from jax.experimental.pallas import tpu_sc as plsc

NH = 8
NP = 4
NL = 4
D = 32
B = 2
LQ = 1024
C = 256
BQ = B * LQ
SHAPES = ((256, 256), (128, 128), (64, 64), (32, 32))
STARTS = (0, 65536, 81920, 86016)
LEN = 87040
NT = 512
NROWS = B * LEN * NH

NC = 2
NS = 16
NW = NC * NS
BQ_PER_W = BQ // NW


def _mm_bias_kernel(x_ref, w_ref, b_ref, o_ref):
    o_ref[...] = jnp.dot(x_ref[...], w_ref[...],
                         preferred_element_type=jnp.float32) + b_ref[...]


def _round_bf16_bits(b):
    return b + 0x7FFF + ((b >> 16) & 1)


def _value_mm_kernel(x_ref, wlo_ref, whi_ref, blo_ref, bhi_ref, o_ref):
    x = x_ref[...]
    lo = jnp.dot(x, wlo_ref[...], preferred_element_type=jnp.float32) + blo_ref[...]
    hi = jnp.dot(x, whi_ref[...], preferred_element_type=jnp.float32) + bhi_ref[...]
    rl = _round_bf16_bits(jax.lax.bitcast_convert_type(lo, jnp.int32))
    rh = _round_bf16_bits(jax.lax.bitcast_convert_type(hi, jnp.int32))
    o_ref[...] = ((rl >> 16) & 0xFFFF) | (rh & jnp.int32(-65536))


def _value_mm(x, wlo, whi, blo, bhi, tile_m):
    m, k = x.shape
    return pl.pallas_call(
        _value_mm_kernel,
        grid=(m // tile_m,),
        in_specs=[
            pl.BlockSpec((tile_m, k), lambda i: (i, 0)),
            pl.BlockSpec((k, 128), lambda i: (0, 0)),
            pl.BlockSpec((k, 128), lambda i: (0, 0)),
            pl.BlockSpec((1, 128), lambda i: (0, 0)),
            pl.BlockSpec((1, 128), lambda i: (0, 0)),
        ],
        out_specs=pl.BlockSpec((tile_m, 128), lambda i: (i, 0)),
        out_shape=jax.ShapeDtypeStruct((m, 128), jnp.int32),
    )(x, wlo, whi, blo[None, :], bhi[None, :])


def _mm_bias(x, wt, b, tile_m):
    m, k = x.shape
    n = wt.shape[1]
    return pl.pallas_call(
        _mm_bias_kernel,
        grid=(m // tile_m,),
        in_specs=[
            pl.BlockSpec((tile_m, k), lambda i: (i, 0)),
            pl.BlockSpec((k, n), lambda i: (0, 0)),
            pl.BlockSpec((1, n), lambda i: (0, 0)),
        ],
        out_specs=pl.BlockSpec((tile_m, n), lambda i: (i, 0)),
        out_shape=jax.ShapeDtypeStruct((m, n), jnp.float32),
    )(x, wt, b[None, :])


def _sample_prep_kernel(q_ref, wx_ref, wy_ref, wa_ref, bx_ref, by_ref, ba_ref,
                        rpx_ref, rpy_ref, idx_ref, w_ref):
    t = q_ref.shape[0]
    q = q_ref[...]
    offx = jnp.dot(q, wx_ref[...], preferred_element_type=jnp.float32) + bx_ref[...]
    offy = jnp.dot(q, wy_ref[...], preferred_element_type=jnp.float32) + by_ref[...]
    logits = jnp.dot(q, wa_ref[...], preferred_element_type=jnp.float32) + ba_ref[...]

    m = jnp.max(logits, axis=1, keepdims=True)
    e = jnp.exp(logits - m)
    gi = lax.broadcasted_iota(jnp.int32, (128, 128), 0) // 16
    gj = lax.broadcasted_iota(jnp.int32, (128, 128), 1) // 16
    gmat = (gi == gj).astype(jnp.float32)
    denom = jnp.dot(e, gmat, preferred_element_type=jnp.float32)
    aw = e / denom

    lane = lax.broadcasted_iota(jnp.int32, (t, 128), 1)
    lvl = (lane % 16) // 4
    hh = lane // 16
    wl_i = jnp.where(lvl == 0, 256, jnp.where(lvl == 1, 128, jnp.where(lvl == 2, 64, 32)))
    hl_i = wl_i
    sl = jnp.where(lvl == 0, 0, jnp.where(lvl == 1, 65536, jnp.where(lvl == 2, 81920, 86016)))
    wl = wl_i.astype(jnp.float32)
    hl = hl_i.astype(jnp.float32)

    row = pl.program_id(0) * t + lax.broadcasted_iota(jnp.int32, (t, 128), 0)
    bvec = row // LQ

    x = rpx_ref[...] * wl + offx - 0.5
    y = rpy_ref[...] * hl + offy - 0.5
    x0 = jnp.floor(x)
    y0 = jnp.floor(y)
    fx = x - x0
    fy = y - y0

    for c, (dx, dy) in enumerate(((0, 0), (1, 0), (0, 1), (1, 1))):
        ix = x0 + dx
        iy = y0 + dy
        valid = ((ix >= 0) & (ix <= wl - 1) & (iy >= 0) & (iy <= hl - 1))
        ixc = jnp.clip(ix, 0, wl - 1).astype(jnp.int32)
        iyc = jnp.clip(iy, 0, hl - 1).astype(jnp.int32)
        pos = sl + iyc * wl_i + ixc
        gidx = (bvec * LEN + pos) * NH + hh
        wb = (fx if dx else 1.0 - fx) * (fy if dy else 1.0 - fy)
        wgt = jnp.where(valid, aw * wb, 0.0)
        idx_ref[c] = gidx
        w_ref[c] = wgt


def _sample_prep(q2, wxt, wyt, wat, bx, by, ba, rpx, rpy):
    tq = 2048
    rep = lambda i: (i, 0)
    full = lambda i: (0, 0)
    return pl.pallas_call(
        _sample_prep_kernel,
        grid=(BQ // tq,),
        in_specs=[
            pl.BlockSpec((tq, C), rep),
            pl.BlockSpec((C, 128), full),
            pl.BlockSpec((C, 128), full),
            pl.BlockSpec((C, 128), full),
            pl.BlockSpec((1, 128), full),
            pl.BlockSpec((1, 128), full),
            pl.BlockSpec((1, 128), full),
            pl.BlockSpec((tq, 128), rep),
            pl.BlockSpec((tq, 128), rep),
        ],
        out_specs=[
            pl.BlockSpec((4, tq, 128), lambda i: (0, i, 0)),
            pl.BlockSpec((4, tq, 128), lambda i: (0, i, 0)),
        ],
        out_shape=[
            jax.ShapeDtypeStruct((4, BQ, 128), jnp.int32),
            jax.ShapeDtypeStruct((4, BQ, 128), jnp.float32),
        ],
    )(q2, wxt, wyt, wat, bx[None, :], by[None, :], ba[None, :], rpx, rpy)


def _lane_bcast(v, j):
    idx = jnp.full((16,), j, dtype=jnp.int32)
    return lax.gather(
        v, idx[:, None],
        dimension_numbers=lax.GatherDimensionNumbers(
            offset_dims=(), collapsed_slice_dims=(0,), start_index_map=(0,)),
        slice_sizes=(1,), mode=lax.GatherScatterMode.PROMISE_IN_BOUNDS)


def _sc_gather_body(table_hbm, idx_hbm, w_hbm, out_hbm,
                    idxs_v, ws_v, rows0_v, rows1_v, out_v, sem0, sem1):
    wid = lax.axis_index("s") * NC + lax.axis_index("c")
    base = wid * BQ_PER_W

    stage = []
    for c in range(4):
        stage.append(pltpu.async_copy(idx_hbm.at[c, pl.ds(base, BQ_PER_W)],
                                      idxs_v.at[c], sem0))
        stage.append(pltpu.async_copy(w_hbm.at[c, pl.ds(base, BQ_PER_W)],
                                      ws_v.at[c], sem0))
    for cp in stage:
        cp.wait()

    def issue(i, rows_v, sem):
        return [pltpu.async_copy(table_hbm.at[idxs_v.at[c, i]],
                                 rows_v.at[pl.ds(c * 128, 128)], sem)
                for c in range(4)]

    def drain(rows_v, sem):
        for c in range(4):
            pltpu.make_async_copy(table_hbm.at[idxs_v.at[0, 0]],
                                  rows_v.at[pl.ds(c * 128, 128)], sem).wait()

    def compute(i, rows_v):
        def h_body(h, carry):
            hbase = h * 16
            acc_e = jnp.zeros((16,), jnp.float32)
            acc_o = jnp.zeros((16,), jnp.float32)
            for c in range(4):
                wv = ws_v[c, i, pl.ds(hbase, 16)]
                for j in range(16):
                    wj = _lane_bcast(wv, j)
                    bits = rows_v[c * 128 + hbase + j, :]
                    even = plsc.bitcast(bits << 16, jnp.float32)
                    odd = plsc.bitcast(bits, jnp.float32)
                    acc_e = acc_e + wj * even
                    acc_o = acc_o + wj * odd
            out_v[i, h, pl.ds(0, 16)] = acc_e
            out_v[i, h, pl.ds(16, 16)] = acc_o
            return carry

        lax.fori_loop(0, NH, h_body, 0)

    issue(0, rows0_v, sem0)

    def step(s, carry):
        i0 = 2 * s
        i1 = i0 + 1
        issue(i1, rows1_v, sem1)
        drain(rows0_v, sem0)
        compute(i0, rows0_v)

        @pl.when(s < BQ_PER_W // 2 - 1)
        def _():
            issue(i0 + 2, rows0_v, sem0)

        drain(rows1_v, sem1)
        compute(i1, rows1_v)
        return carry

    lax.fori_loop(0, BQ_PER_W // 2, step, 0)
    pltpu.sync_copy(out_v, out_hbm.at[pl.ds(base, BQ_PER_W)])


def _sc_gather(table, idx3, w3):
    mesh = plsc.VectorSubcoreMesh(core_axis_name="c", subcore_axis_name="s")
    f = functools.partial(
        pl.kernel, _sc_gather_body, mesh=mesh,
        compiler_params=pltpu.CompilerParams(use_tc_tiling_on_sc=False,
                                             needs_layout_passes=False),
        out_type=jax.ShapeDtypeStruct((BQ, NH, D), jnp.float32),
        scratch_types=[
            pltpu.VMEM((4, BQ_PER_W, 128), jnp.int32),
            pltpu.VMEM((4, BQ_PER_W, 128), jnp.float32),
            pltpu.VMEM((NT, D // 2), jnp.int32),
            pltpu.VMEM((NT, D // 2), jnp.int32),
            pltpu.VMEM((BQ_PER_W, NH, D), jnp.float32),
            pltpu.SemaphoreType.DMA,
            pltpu.SemaphoreType.DMA,
        ],
    )()
    return f(table, idx3, w3)


def kernel(query, reference_points, input_flatten, input_spatial_shapes,
           input_level_start_index, W_value, b_value, W_off, b_off,
           W_attn, b_attn, W_out, b_out):
    q2 = query.reshape(BQ, C)

    wvt = W_value.T
    ch = jnp.arange(C)
    cols_lo = (ch // 16) * 32 + ch % 16
    cols = jnp.concatenate([cols_lo[:128], cols_lo[:128] + 16])
    wlo = wvt[:, cols[:128]]
    whi = wvt[:, cols[128:]]
    value = _value_mm(input_flatten.reshape(B * LEN, C), wlo, whi,
                      b_value[cols[:128]], b_value[cols[128:]], 10880)
    table = value.reshape(NROWS, D // 2)

    wxt = W_off[0::2].T
    wyt = W_off[1::2].T
    bx = b_off[0::2]
    by = b_off[1::2]
    rp = reference_points.reshape(BQ, NL, 2)
    rpx = jnp.tile(jnp.repeat(rp[:, :, 0], NP, axis=1), (1, NH))
    rpy = jnp.tile(jnp.repeat(rp[:, :, 1], NP, axis=1), (1, NH))
    idx_all, w_all = _sample_prep(q2, wxt, wyt, W_attn.T, bx, by, b_attn, rpx, rpy)

    acc = _sc_gather(table, idx_all, w_all)

    out = _mm_bias(acc.reshape(BQ, C), W_out.T, b_out, 1024)
    return out.reshape(B, LQ, C)
